# baseline (speedup 1.0000x reference)
import functools

import jax
import jax.numpy as jnp
from jax.experimental import pallas as pl
from jax.experimental.pallas import tpu as pltpu

F32 = jnp.float32
BF16 = jnp.bfloat16
PAD = 128


def _ln(x, scale, bias, eps=1e-6):
    mu = jnp.mean(x, axis=-1, keepdims=True)
    var = jnp.mean((x - mu) ** 2, axis=-1, keepdims=True)
    return (x - mu) / jnp.sqrt(var + eps) * scale + bias


def _stage_a_kernel(x_ref, s_ref, b_ref, xhp_ref, pooled_ref, *, H, dh):
    xb = x_ref[0]
    normed = _ln(xb, s_ref[...], b_ref[...])
    pooled_ref[0] = jnp.mean(normed, axis=0, keepdims=True)
    nb = normed.astype(BF16)
    N = nb.shape[0]
    zpad = jnp.zeros((N, PAD - dh), BF16)
    pieces = [jnp.concatenate([nb[:, h * dh:(h + 1) * dh], zpad], axis=1)
              for h in range(H)]
    xhp_ref[...] = jnp.concatenate(pieces, axis=0)


def _stage_a2_kernel(pooled_ref, rw_ref, rb_ref, pw_ref, w1_ref, w2_ref,
                     gmat_ref, aux_ref, pwb_ref, w1b_ref, w2b_ref):
    B, E = pooled_ref.shape[0], rw_ref.shape[1]
    logits = jnp.dot(pooled_ref[...].astype(BF16), rw_ref[...].astype(BF16),
                     preferred_element_type=F32) + rb_ref[...]
    m = jnp.max(logits, axis=-1, keepdims=True)
    ex = jnp.exp(logits - m)
    probs = ex / jnp.sum(ex, axis=-1, keepdims=True)
    iota = jax.lax.broadcasted_iota(jnp.int32, (B, E), 1)
    v1 = jnp.max(probs, axis=-1, keepdims=True)
    i1 = jnp.min(jnp.where(probs == v1, iota, E), axis=-1, keepdims=True)
    masked = jnp.where(iota == i1, -jnp.inf, probs)
    v2 = jnp.max(masked, axis=-1, keepdims=True)
    i2 = jnp.min(jnp.where(masked == v2, iota, E), axis=-1, keepdims=True)
    s = v1 + v2
    gmat_ref[...] = ((iota == i1).astype(F32) * (v1 / s)
                     + (iota == i2).astype(F32) * (v2 / s))
    cnt = (iota == i1).astype(F32) + (iota == i2).astype(F32)
    frac = jnp.sum(cnt, axis=0, keepdims=True) / (B * 2)
    mean_p = jnp.mean(probs, axis=0, keepdims=True)
    aux_ref[...] = E * jnp.sum(frac * mean_p, keepdims=True)
    pwb_ref[...] = pw_ref[...].astype(BF16)
    w1b_ref[...] = w1_ref[...].astype(BF16)
    w2b_ref[...] = w2_ref[...].astype(BF16)


def _stage_bc_kernel(g_ref, w_ref, xhp_ref, x_ref, pwb_ref, pb_ref,
                     s2_ref, b2_ref, w1b_ref, b1b_ref, w2b_ref, b2b_ref,
                     out_ref, scr_ref, n2_ref, *, H, dh, B, E, N, NJ):
    i = pl.program_id(0)
    H2 = H // 2
    NMIX = 2 * E

    @pl.when(i == 0)
    def _():
        scr_ref[...] = jnp.zeros_like(scr_ref)

    def mix_phase(s):
        e = i - s * E
        gsum = 0.0
        for b in range(B):
            gsum += g_ref[b * E + e]

        @pl.when(gsum > 0.0)
        def _():
            wcast = [w_ref[0, hl].astype(BF16) for hl in range(H2)]
            for b in range(B):
                g = g_ref[b * E + e]

                @pl.when(g > 0.0)
                def _():
                    for hl in range(H2):
                        xs = xhp_ref[hl * N:(hl + 1) * N,
                                     b * PAD:(b + 1) * PAD]
                        y = jnp.dot(wcast[hl], xs,
                                    preferred_element_type=F32)
                        scr_ref[s, b, hl * N:(hl + 1) * N, :] += y * g

    @pl.when(i < E)
    def _():
        mix_phase(0)

    @pl.when(jnp.logical_and(i >= E, i < NMIX))
    def _():
        mix_phase(1)

    @pl.when(i >= NMIX)
    def _():
        t = i - NMIX
        b = t // NJ
        j = t - b * NJ

        @pl.when(j == 0)
        def _():
            lo = scr_ref[0, b]
            hi = scr_ref[1, b]
            pieces = ([lo[hl * N:(hl + 1) * N, 0:dh] for hl in range(H2)]
                      + [hi[hl * N:(hl + 1) * N, 0:dh] for hl in range(H2)])
            mixed = jnp.concatenate(pieces, axis=1)
            u = x_ref[0] + jnp.dot(mixed.astype(BF16), pwb_ref[...],
                                   preferred_element_type=F32) + pb_ref[...]
            n2_ref[...] = _ln(u, s2_ref[...], b2_ref[...]).astype(BF16)
            out_ref[0] = u + b2b_ref[...]

        h1 = jnp.dot(n2_ref[...], w1b_ref[...],
                     preferred_element_type=F32) + b1b_ref[...]
        h1 = (0.5 * h1 * (1.0 + jax.lax.erf(h1 * 0.7071067811865476)))
        out_ref[0] += jnp.dot(h1.astype(BF16), w2b_ref[...],
                              preferred_element_type=F32)


def kernel(x, ln1_scale, ln1_bias, router_w, router_b, expert_w, proj_w,
           proj_b, ln2_scale, ln2_bias, mlp_w1, mlp_b1, mlp_w2, mlp_b2):
    B, N, D = x.shape
    E, H = expert_w.shape[0], expert_w.shape[1]
    dh = D // H
    hid = mlp_w1.shape[1]
    HN = H * N
    HN2 = HN // 2
    NJ = 4
    HIDC = hid // NJ
    NMIX = 2 * E

    xhp, pooled = pl.pallas_call(
        functools.partial(_stage_a_kernel, H=H, dh=dh),
        grid=(B,),
        in_specs=[
            pl.BlockSpec((1, N, D), lambda b: (b, 0, 0)),
            pl.BlockSpec((1, D), lambda b: (0, 0)),
            pl.BlockSpec((1, D), lambda b: (0, 0)),
        ],
        out_specs=[
            pl.BlockSpec((HN, PAD), lambda b: (0, b)),
            pl.BlockSpec((1, 1, D), lambda b: (b, 0, 0)),
        ],
        out_shape=[
            jax.ShapeDtypeStruct((HN, B * PAD), BF16),
            jax.ShapeDtypeStruct((B, 1, D), F32),
        ],
    )(x, ln1_scale.reshape(1, D), ln1_bias.reshape(1, D))
    pooled = pooled.reshape(B, D)

    gmat, aux, pwb, w1b, w2b = pl.pallas_call(
        _stage_a2_kernel,
        out_shape=[
            jax.ShapeDtypeStruct((B, E), F32),
            jax.ShapeDtypeStruct((1, 1), F32),
            jax.ShapeDtypeStruct((D, D), BF16),
            jax.ShapeDtypeStruct((D, hid), BF16),
            jax.ShapeDtypeStruct((hid, D), BF16),
        ],
    )(pooled, router_w, router_b.reshape(1, E), proj_w, mlp_w1, mlp_w2)

    def w_idx(i, g):
        mixing = i < NMIX
        e = jnp.where(mixing, jnp.where(i < E, i, i - E), E - 1)
        s = jnp.where(i < E, 0, 1)
        return (e, s, 0, 0)

    def xhp_idx(i, g):
        return (jnp.where(i < E, 0, 1), 0)

    def b_of(i):
        return jnp.where(i < NMIX, 0, (i - NMIX) // NJ)

    def j_of(i):
        return jnp.where(i < NMIX, 0, (i - NMIX) % NJ)

    y = pl.pallas_call(
        functools.partial(_stage_bc_kernel, H=H, dh=dh, B=B, E=E, N=N, NJ=NJ),
        grid_spec=pltpu.PrefetchScalarGridSpec(
            num_scalar_prefetch=1,
            grid=(NMIX + B * NJ,),
            in_specs=[
                pl.BlockSpec((1, H // 2, N, N), w_idx),
                pl.BlockSpec((HN2, B * PAD), xhp_idx),
                pl.BlockSpec((1, N, D), lambda i, g: (b_of(i), 0, 0)),
                pl.BlockSpec((D, D), lambda i, g: (0, 0)),
                pl.BlockSpec((1, D), lambda i, g: (0, 0)),
                pl.BlockSpec((1, D), lambda i, g: (0, 0)),
                pl.BlockSpec((1, D), lambda i, g: (0, 0)),
                pl.BlockSpec((D, HIDC), lambda i, g: (0, j_of(i))),
                pl.BlockSpec((1, HIDC), lambda i, g: (0, j_of(i))),
                pl.BlockSpec((HIDC, D), lambda i, g: (j_of(i), 0)),
                pl.BlockSpec((1, D), lambda i, g: (0, 0)),
            ],
            out_specs=pl.BlockSpec((1, N, D), lambda i, g: (b_of(i), 0, 0)),
            scratch_shapes=[
                pltpu.VMEM((2, B, HN2, PAD), F32),
                pltpu.VMEM((N, D), BF16),
            ],
        ),
        out_shape=jax.ShapeDtypeStruct((B, N, D), F32),
    )(gmat.reshape(B * E), expert_w,
      xhp, x, pwb, proj_b.reshape(1, D), ln2_scale.reshape(1, D),
      ln2_bias.reshape(1, D), w1b, mlp_b1.reshape(1, hid), w2b,
      mlp_b2.reshape(1, D))

    return (y, aux.reshape(()))


# speedup vs baseline: 1.0450x; 1.0450x over previous
import functools

import jax
import jax.numpy as jnp
from jax.experimental import pallas as pl
from jax.experimental.pallas import tpu as pltpu

F32 = jnp.float32
BF16 = jnp.bfloat16


def _ln(x, scale, bias, eps=1e-6):
    mu = jnp.mean(x, axis=-1, keepdims=True)
    var = jnp.mean((x - mu) ** 2, axis=-1, keepdims=True)
    return (x - mu) / jnp.sqrt(var + eps) * scale + bias


def _stage_a_kernel(x_ref, s_ref, b_ref, normed_ref, pooled_ref):
    xb = x_ref[0]
    normed = _ln(xb, s_ref[...], b_ref[...])
    normed_ref[0] = normed
    pooled_ref[0] = jnp.mean(normed, axis=0, keepdims=True)


def _stage_a2_kernel(pooled_ref, rw_ref, rb_ref, gmat_ref, aux_ref):
    B, E = pooled_ref.shape[0], rw_ref.shape[1]
    logits = jnp.dot(pooled_ref[...].astype(BF16), rw_ref[...].astype(BF16),
                     preferred_element_type=F32) + rb_ref[...]
    m = jnp.max(logits, axis=-1, keepdims=True)
    ex = jnp.exp(logits - m)
    probs = ex / jnp.sum(ex, axis=-1, keepdims=True)
    iota = jax.lax.broadcasted_iota(jnp.int32, (B, E), 1)
    v1 = jnp.max(probs, axis=-1, keepdims=True)
    i1 = jnp.min(jnp.where(probs == v1, iota, E), axis=-1, keepdims=True)
    masked = jnp.where(iota == i1, -jnp.inf, probs)
    v2 = jnp.max(masked, axis=-1, keepdims=True)
    i2 = jnp.min(jnp.where(masked == v2, iota, E), axis=-1, keepdims=True)
    s = v1 + v2
    gmat_ref[...] = ((iota == i1).astype(F32) * (v1 / s)
                     + (iota == i2).astype(F32) * (v2 / s))
    cnt = (iota == i1).astype(F32) + (iota == i2).astype(F32)
    frac = jnp.sum(cnt, axis=0, keepdims=True) / (B * 2)
    mean_p = jnp.mean(probs, axis=0, keepdims=True)
    aux_ref[...] = E * jnp.sum(frac * mean_p, keepdims=True)


def _stage_b_kernel(g_ref, w_ref, x_ref, out_ref, *, H, dh, B, E):
    e = pl.program_id(0)

    @pl.when(e == 0)
    def _():
        out_ref[...] = jnp.zeros_like(out_ref)

    gsum = 0.0
    for b in range(B):
        gsum += g_ref[b * E + e]

    @pl.when(gsum > 0.0)
    def _():
        wcast = [w_ref[0, h].astype(BF16) for h in range(H)]
        for b in range(B):
            g = g_ref[b * E + e]

            @pl.when(g > 0.0)
            def _():
                xb = x_ref[b]
                pieces = []
                for h in range(H):
                    xs = xb[:, h * dh:(h + 1) * dh].astype(BF16)
                    pieces.append(jnp.dot(wcast[h], xs,
                                          preferred_element_type=F32))
                out_ref[b] += jnp.concatenate(pieces, axis=1) * g


def _stage_c_kernel(x_ref, m_ref, pw_ref, pb_ref, s2_ref, b2_ref,
                    w1_ref, b1_ref, w2_ref, b2b_ref, out_ref, *, hid_chunk):
    u = x_ref[...] + jnp.dot(m_ref[...].astype(BF16), pw_ref[...].astype(BF16),
                             preferred_element_type=F32) + pb_ref[...]
    n2 = _ln(u, s2_ref[...], b2_ref[...]).astype(BF16)
    hid = w1_ref.shape[1]
    acc = u + b2b_ref[...]
    for j in range(0, hid, hid_chunk):
        h1 = jnp.dot(n2, w1_ref[:, j:j + hid_chunk].astype(BF16),
                     preferred_element_type=F32) + b1_ref[:, j:j + hid_chunk]
        h1 = (0.5 * h1 * (1.0 + jax.lax.erf(h1 * 0.7071067811865476)))
        acc = acc + jnp.dot(h1.astype(BF16), w2_ref[j:j + hid_chunk, :].astype(BF16),
                            preferred_element_type=F32)
    out_ref[...] = acc


def kernel(x, ln1_scale, ln1_bias, router_w, router_b, expert_w, proj_w,
           proj_b, ln2_scale, ln2_bias, mlp_w1, mlp_b1, mlp_w2, mlp_b2):
    B, N, D = x.shape
    E, H = expert_w.shape[0], expert_w.shape[1]
    dh = D // H
    hid = mlp_w1.shape[1]

    normed, pooled = pl.pallas_call(
        _stage_a_kernel,
        grid=(B,),
        in_specs=[
            pl.BlockSpec((1, N, D), lambda b: (b, 0, 0)),
            pl.BlockSpec((1, D), lambda b: (0, 0)),
            pl.BlockSpec((1, D), lambda b: (0, 0)),
        ],
        out_specs=[
            pl.BlockSpec((1, N, D), lambda b: (b, 0, 0)),
            pl.BlockSpec((1, 1, D), lambda b: (b, 0, 0)),
        ],
        out_shape=[
            jax.ShapeDtypeStruct((B, N, D), F32),
            jax.ShapeDtypeStruct((B, 1, D), F32),
        ],
    )(x, ln1_scale.reshape(1, D), ln1_bias.reshape(1, D))
    pooled = pooled.reshape(B, D)

    gmat, aux = pl.pallas_call(
        _stage_a2_kernel,
        out_shape=[
            jax.ShapeDtypeStruct((B, E), F32),
            jax.ShapeDtypeStruct((1, 1), F32),
        ],
    )(pooled, router_w, router_b.reshape(1, E))

    mixed = pl.pallas_call(
        functools.partial(_stage_b_kernel, H=H, dh=dh, B=B, E=E),
        grid_spec=pltpu.PrefetchScalarGridSpec(
            num_scalar_prefetch=1,
            grid=(E,),
            in_specs=[
                pl.BlockSpec((1, H, N, N), lambda e, g: (e, 0, 0, 0)),
                pl.BlockSpec((B, N, D), lambda e, g: (0, 0, 0)),
            ],
            out_specs=pl.BlockSpec((B, N, D), lambda e, g: (0, 0, 0)),
        ),
        out_shape=jax.ShapeDtypeStruct((B, N, D), F32),
    )(gmat.reshape(B * E), expert_w, normed)

    R = 256
    rows = B * N
    y = pl.pallas_call(
        functools.partial(_stage_c_kernel, hid_chunk=1536),
        grid=(rows // R,),
        in_specs=[
            pl.BlockSpec((R, D), lambda r: (r, 0)),
            pl.BlockSpec((R, D), lambda r: (r, 0)),
            pl.BlockSpec((D, D), lambda r: (0, 0)),
            pl.BlockSpec((1, D), lambda r: (0, 0)),
            pl.BlockSpec((1, D), lambda r: (0, 0)),
            pl.BlockSpec((1, D), lambda r: (0, 0)),
            pl.BlockSpec((D, hid), lambda r: (0, 0)),
            pl.BlockSpec((1, hid), lambda r: (0, 0)),
            pl.BlockSpec((hid, D), lambda r: (0, 0)),
            pl.BlockSpec((1, D), lambda r: (0, 0)),
        ],
        out_specs=pl.BlockSpec((R, D), lambda r: (r, 0)),
        out_shape=jax.ShapeDtypeStruct((rows, D), F32),
    )(x.reshape(rows, D), mixed.reshape(rows, D), proj_w,
      proj_b.reshape(1, D), ln2_scale.reshape(1, D), ln2_bias.reshape(1, D),
      mlp_w1, mlp_b1.reshape(1, hid), mlp_w2, mlp_b2.reshape(1, D))

    return (y.reshape(B, N, D), aux.reshape(()))


# speedup vs baseline: 1.1662x; 1.1159x over previous
import functools

import jax
import jax.numpy as jnp
from jax.experimental import pallas as pl
from jax.experimental.pallas import tpu as pltpu

F32 = jnp.float32
BF16 = jnp.bfloat16


def _ln(x, scale, bias, eps=1e-6):
    mu = jnp.mean(x, axis=-1, keepdims=True)
    var = jnp.mean((x - mu) ** 2, axis=-1, keepdims=True)
    return (x - mu) / jnp.sqrt(var + eps) * scale + bias


def _stage_a_kernel(x_ref, s_ref, b_ref, normed_ref, pooled_ref):
    xb = x_ref[0]
    normed = _ln(xb, s_ref[...], b_ref[...])
    normed_ref[0] = normed
    pooled_ref[0] = jnp.mean(normed, axis=0, keepdims=True)


def _stage_a2_kernel(pooled_ref, rw_ref, rb_ref, gmat_ref, aux_ref):
    B, E = pooled_ref.shape[0], rw_ref.shape[1]
    logits = jnp.dot(pooled_ref[...].astype(BF16), rw_ref[...].astype(BF16),
                     preferred_element_type=F32) + rb_ref[...]
    m = jnp.max(logits, axis=-1, keepdims=True)
    ex = jnp.exp(logits - m)
    probs = ex / jnp.sum(ex, axis=-1, keepdims=True)
    iota = jax.lax.broadcasted_iota(jnp.int32, (B, E), 1)
    v1 = jnp.max(probs, axis=-1, keepdims=True)
    i1 = jnp.min(jnp.where(probs == v1, iota, E), axis=-1, keepdims=True)
    masked = jnp.where(iota == i1, -jnp.inf, probs)
    v2 = jnp.max(masked, axis=-1, keepdims=True)
    i2 = jnp.min(jnp.where(masked == v2, iota, E), axis=-1, keepdims=True)
    s = v1 + v2
    gmat_ref[...] = ((iota == i1).astype(F32) * (v1 / s)
                     + (iota == i2).astype(F32) * (v2 / s))
    cnt = (iota == i1).astype(F32) + (iota == i2).astype(F32)
    frac = jnp.sum(cnt, axis=0, keepdims=True) / (B * 2)
    mean_p = jnp.mean(probs, axis=0, keepdims=True)
    aux_ref[...] = E * jnp.sum(frac * mean_p, keepdims=True)


def _stage_b_kernel(g_ref, w_ref, x_ref, out_ref, *, H, dh, B, E):
    e = pl.program_id(0)

    @pl.when(e == 0)
    def _():
        out_ref[...] = jnp.zeros_like(out_ref)

    for b in range(B):
        g = g_ref[b * E + e]

        @pl.when(g > 0.0)
        def _():
            xb = x_ref[b]
            pieces = []
            for h in range(H):
                w = w_ref[0, h].astype(BF16)
                xs = xb[:, h * dh:(h + 1) * dh].astype(BF16)
                pieces.append(jnp.dot(w, xs, preferred_element_type=F32))
            out_ref[b] += jnp.concatenate(pieces, axis=1) * g


def _stage_c_kernel(x_ref, m_ref, pw_ref, pb_ref, s2_ref, b2_ref,
                    w1_ref, b1_ref, w2_ref, b2b_ref, out_ref, *, hid_chunk):
    u = x_ref[...] + jnp.dot(m_ref[...].astype(BF16), pw_ref[...].astype(BF16),
                             preferred_element_type=F32) + pb_ref[...]
    n2 = _ln(u, s2_ref[...], b2_ref[...]).astype(BF16)
    hid = w1_ref.shape[1]
    acc = u + b2b_ref[...]
    for j in range(0, hid, hid_chunk):
        h1 = jnp.dot(n2, w1_ref[:, j:j + hid_chunk].astype(BF16),
                     preferred_element_type=F32) + b1_ref[:, j:j + hid_chunk]
        h1 = (0.5 * h1 * (1.0 + jax.lax.erf(h1 * 0.7071067811865476))).astype(BF16)
        acc = acc + jnp.dot(h1, w2_ref[j:j + hid_chunk, :].astype(BF16),
                            preferred_element_type=F32)
    out_ref[...] = acc


def kernel(x, ln1_scale, ln1_bias, router_w, router_b, expert_w, proj_w,
           proj_b, ln2_scale, ln2_bias, mlp_w1, mlp_b1, mlp_w2, mlp_b2):
    B, N, D = x.shape
    E, H = expert_w.shape[0], expert_w.shape[1]
    dh = D // H
    hid = mlp_w1.shape[1]

    normed, pooled = pl.pallas_call(
        _stage_a_kernel,
        grid=(B,),
        in_specs=[
            pl.BlockSpec((1, N, D), lambda b: (b, 0, 0)),
            pl.BlockSpec((1, D), lambda b: (0, 0)),
            pl.BlockSpec((1, D), lambda b: (0, 0)),
        ],
        out_specs=[
            pl.BlockSpec((1, N, D), lambda b: (b, 0, 0)),
            pl.BlockSpec((1, 1, D), lambda b: (b, 0, 0)),
        ],
        out_shape=[
            jax.ShapeDtypeStruct((B, N, D), F32),
            jax.ShapeDtypeStruct((B, 1, D), F32),
        ],
    )(x, ln1_scale.reshape(1, D), ln1_bias.reshape(1, D))
    pooled = pooled.reshape(B, D)

    gmat, aux = pl.pallas_call(
        _stage_a2_kernel,
        out_shape=[
            jax.ShapeDtypeStruct((B, E), F32),
            jax.ShapeDtypeStruct((1, 1), F32),
        ],
    )(pooled, router_w, router_b.reshape(1, E))

    mixed = pl.pallas_call(
        functools.partial(_stage_b_kernel, H=H, dh=dh, B=B, E=E),
        grid_spec=pltpu.PrefetchScalarGridSpec(
            num_scalar_prefetch=1,
            grid=(E,),
            in_specs=[
                pl.BlockSpec((1, H, N, N), lambda e, g: (e, 0, 0, 0)),
                pl.BlockSpec((B, N, D), lambda e, g: (0, 0, 0)),
            ],
            out_specs=pl.BlockSpec((B, N, D), lambda e, g: (0, 0, 0)),
        ),
        out_shape=jax.ShapeDtypeStruct((B, N, D), F32),
    )(gmat.reshape(B * E), expert_w, normed)

    R = 512
    rows = B * N
    y = pl.pallas_call(
        functools.partial(_stage_c_kernel, hid_chunk=768),
        grid=(rows // R,),
        in_specs=[
            pl.BlockSpec((R, D), lambda r: (r, 0)),
            pl.BlockSpec((R, D), lambda r: (r, 0)),
            pl.BlockSpec((D, D), lambda r: (0, 0)),
            pl.BlockSpec((1, D), lambda r: (0, 0)),
            pl.BlockSpec((1, D), lambda r: (0, 0)),
            pl.BlockSpec((1, D), lambda r: (0, 0)),
            pl.BlockSpec((D, hid), lambda r: (0, 0)),
            pl.BlockSpec((1, hid), lambda r: (0, 0)),
            pl.BlockSpec((hid, D), lambda r: (0, 0)),
            pl.BlockSpec((1, D), lambda r: (0, 0)),
        ],
        out_specs=pl.BlockSpec((R, D), lambda r: (r, 0)),
        out_shape=jax.ShapeDtypeStruct((rows, D), F32),
    )(x.reshape(rows, D), mixed.reshape(rows, D), proj_w,
      proj_b.reshape(1, D), ln2_scale.reshape(1, D), ln2_bias.reshape(1, D),
      mlp_w1, mlp_b1.reshape(1, hid), mlp_w2, mlp_b2.reshape(1, D))

    return (y.reshape(B, N, D), aux.reshape(()))
